# Initial kernel scaffold; baseline (speedup 1.0000x reference)
import jax
import jax.numpy as jnp
from jax.experimental import pallas as pl


def kernel(x, adj, W1, b1, W2, b2, Wg, a):
    raise NotImplementedError("write your pallas kernel here")


# speedup vs baseline: 1.5040x; 1.5040x over previous
import jax
import jax.numpy as jnp
from jax import lax
from jax.experimental import pallas as pl

N = 1024
IN_DIM = 512
HID = 256
OUT = 128


def _leaky(v, slope=0.25):
    return jnp.where(v >= 0, v, slope * v)


def _encoder_body(x_ref, adj_ref, w1_ref, b1_ref, w2_ref, b2_ref, wg_ref,
                  a1_ref, a2_ref, out_ref):
    adj = adj_ref[...]
    f32 = jnp.float32
    s1 = jnp.dot(x_ref[...], w1_ref[...], preferred_element_type=f32)
    x1 = _leaky(jnp.dot(adj, s1, preferred_element_type=f32) + b1_ref[...])
    s2 = jnp.dot(x1, w2_ref[...], preferred_element_type=f32)
    x2 = _leaky(jnp.dot(adj, s2, preferred_element_type=f32) + b2_ref[...])
    h = jnp.dot(x2, wg_ref[...], preferred_element_type=f32)
    ha1 = jnp.sum(h * a1_ref[...], axis=1, keepdims=True)
    ha2 = lax.dot_general(a2_ref[...], h, (((1,), (1,)), ((), ())),
                          preferred_element_type=f32)
    e = _leaky(ha1 + ha2)
    att = jnp.where(adj > 0, e, jnp.float32(-1e12))
    att = att - jnp.max(att, axis=1, keepdims=True)
    p = jnp.exp(att)
    att = p / jnp.sum(p, axis=1, keepdims=True)
    out_ref[...] = _leaky(jnp.dot(att, h, preferred_element_type=f32))


def kernel(x, adj, W1, b1, W2, b2, Wg, a):
    out = pl.pallas_call(
        _encoder_body,
        out_shape=jax.ShapeDtypeStruct((N, OUT), jnp.float32),
    )(x, adj, W1, b1.reshape(1, HID), W2, b2.reshape(1, HID), Wg,
      a[:OUT].reshape(1, OUT), a[OUT:].reshape(1, OUT))
    return out[:, : OUT // 2], out[:, OUT // 2:]
